# Initial kernel scaffold; baseline (speedup 1.0000x reference)
import jax
import jax.numpy as jnp
from jax.experimental import pallas as pl


def kernel(x, conv1_w, conv1_b, conv2_w, conv2_b, conv3_w, conv3_b, fc1_w, fc1_b, fc2_w, fc2_b, fc3_w, fc3_b, batch, batch_size, nr_points):
    raise NotImplementedError("write your pallas kernel here")


# speedup vs baseline: 1.7638x; 1.7638x over previous
import functools

import jax
import jax.numpy as jnp
from jax.experimental import pallas as pl
from jax.experimental.pallas import tpu as pltpu

_F32 = jnp.float32


_BF16 = jnp.bfloat16


def _bdot(a, b, dims):
    return jax.lax.dot_general(a.astype(_BF16), b.astype(_BF16), dims,
                               preferred_element_type=_F32)


def _xdot(a, b, dims):
    return jax.lax.dot_general(a, b, dims, precision=jax.lax.Precision.HIGHEST,
                               preferred_element_type=_F32)


def _graph_cheb(X, wk_ref, bias_ref, K):
    N, F = X.shape

    Xsq = X * X
    sq_col = jnp.sum(Xsq, axis=1, keepdims=True)
    ones_row = jnp.ones((1, F), _F32)
    sq_row = _xdot(ones_row, Xsq, (((1,), (1,)), ((), ())))
    inner = -2.0 * _bdot(X, X, (((1,), (1,)), ((), ())))
    adj = sq_col + inner + sq_row
    Wg = jnp.exp(-adj)

    rows = jax.lax.broadcasted_iota(jnp.int32, (N, N), 0)
    cols = jax.lax.broadcasted_iota(jnp.int32, (N, N), 1)
    diag = rows == cols
    A = jnp.where(diag, 0.0, Wg)

    d_col = jnp.sum(A, axis=1, keepdims=True)
    d_row = jnp.sum(A, axis=0, keepdims=True)
    dinv_col = jnp.where(d_col > 0, 1.0 / jnp.sqrt(jnp.where(d_col > 0, d_col, 1.0)), 0.0)
    dinv_row = jnp.where(d_row > 0, 1.0 / jnp.sqrt(jnp.where(d_row > 0, d_row, 1.0)), 0.0)
    An = A * dinv_col * dinv_row
    L = jnp.where(diag, 1.0, -An)

    Tx0 = X
    out = _bdot(Tx0, wk_ref[0], (((1,), (0,)), ((), ())))
    if K > 1:
        Tx1 = -_bdot(An, Tx0, (((1,), (0,)), ((), ())))
        out = out + _bdot(Tx1, wk_ref[1], (((1,), (0,)), ((), ())))
        for k in range(2, K):
            Tx2 = -2.0 * _bdot(An, Tx1, (((1,), (0,)), ((), ()))) - Tx0
            out = out + _bdot(Tx2, wk_ref[k], (((1,), (0,)), ((), ())))
            Tx0, Tx1 = Tx1, Tx2
    out = jnp.maximum(out + bias_ref[...], 0.0)
    return out, An, L


def _mreg_update(out, L, mreg, reg_ref, b, nb):
    Lout = _bdot(L, out, (((1,), (0,)), ((), ())))
    contrib = _bdot(out, Lout, (((0,), (0,)), ((), ())))

    @pl.when(b == 0)
    def _():
        mreg[...] = contrib

    @pl.when(b > 0)
    def _():
        mreg[...] = mreg[...] + contrib

    @pl.when(b == nb - 1)
    def _():
        m = mreg[...]
        reg_ref[...] = jnp.broadcast_to(jnp.sqrt(jnp.sum(m * m)), (1, 1))


def _layer_body(x_ref, wk_ref, bias_ref, out_ref, reg_ref, mreg, *, K, nb):
    b = pl.program_id(0)
    out, _, L = _graph_cheb(x_ref[0], wk_ref, bias_ref, K)
    out_ref[0] = out
    _mreg_update(out, L, mreg, reg_ref, b, nb)


def _layer3_body(x_ref, wk_ref, bias_ref, pooled_ref, reg_ref, mreg, *, K, nb):
    b = pl.program_id(0)
    out, _, L = _graph_cheb(x_ref[0], wk_ref, bias_ref, K)
    pooled_ref[0] = jnp.max(out, axis=0, keepdims=True)
    _mreg_update(out, L, mreg, reg_ref, b, nb)


def _head_body(p_ref, w1_ref, b1_ref, w2_ref, b2_ref, w3_ref, b3_ref,
               logits_ref, tail_ref):
    mm = lambda a, w: _bdot(a, w, (((1,), (0,)), ((), ())))
    h = jnp.maximum(mm(p_ref[...], w1_ref[...]) + b1_ref[...], 0.0)
    h = jnp.maximum(mm(h, w2_ref[...]) + b2_ref[...], 0.0)
    logits_ref[...] = mm(h, w3_ref[...]) + b3_ref[...]
    w1 = w1_ref[...]
    nw = jnp.sqrt(jnp.sum(w1 * w1))
    b1 = b1_ref[...]
    nb = jnp.sqrt(jnp.sum(b1 * b1))
    lane = jax.lax.broadcasted_iota(jnp.int32, (1, 8), 1)
    tail_ref[...] = jnp.where(lane % 2 == 0,
                              jnp.broadcast_to(nw, (1, 8)),
                              jnp.broadcast_to(nb, (1, 8)))


def _run_layer(x, wk, bias, last):
    B, N, Fin = x.shape
    K, _, Fout = wk.shape
    bias2 = bias.reshape(1, Fout)
    body = _layer3_body if last else _layer_body
    out_specs = [
        pl.BlockSpec((1, 1, Fout) if last else (1, N, Fout),
                     lambda b: (b, 0, 0)),
        pl.BlockSpec((1, 1), lambda b: (0, 0)),
    ]
    out_shape = [
        jax.ShapeDtypeStruct((B, 1, Fout) if last else (B, N, Fout), _F32),
        jax.ShapeDtypeStruct((1, 1), _F32),
    ]
    return pl.pallas_call(
        functools.partial(body, K=K, nb=B),
        grid=(B,),
        in_specs=[
            pl.BlockSpec((1, N, Fin), lambda b: (b, 0, 0)),
            pl.BlockSpec((K, Fin, Fout), lambda b: (0, 0, 0)),
            pl.BlockSpec((1, Fout), lambda b: (0, 0)),
        ],
        out_specs=out_specs,
        out_shape=out_shape,
        scratch_shapes=[pltpu.VMEM((Fout, Fout), _F32)],
        compiler_params=pltpu.CompilerParams(
            dimension_semantics=("arbitrary",)),
    )(x, wk, bias2)


def kernel(x, conv1_w, conv1_b, conv2_w, conv2_b, conv3_w, conv3_b,
           fc1_w, fc1_b, fc2_w, fc2_b, fc3_w, fc3_b,
           batch, batch_size, nr_points):
    del batch, batch_size, nr_points
    out1, r1 = _run_layer(x, conv1_w, conv1_b, last=False)
    out2, r2 = _run_layer(out1, conv2_w, conv2_b, last=False)
    pooled, r3 = _run_layer(out2, conv3_w, conv3_b, last=True)
    pooled = pooled.reshape(pooled.shape[0], pooled.shape[2])

    Bn = pooled.shape[0]
    logits, tail = pl.pallas_call(
        _head_body,
        out_shape=[
            jax.ShapeDtypeStruct((Bn, fc3_w.shape[1]), _F32),
            jax.ShapeDtypeStruct((1, 8), _F32),
        ],
    )(pooled, fc1_w, fc1_b.reshape(1, -1), fc2_w, fc2_b.reshape(1, -1),
      fc3_w, fc3_b.reshape(1, -1))

    regs = jnp.concatenate([
        r1.reshape(1), r2.reshape(1), r3.reshape(1), tail[0, :6]])
    return logits, regs
